# initial kernel scaffold (unmeasured)
import jax
import jax.numpy as jnp
from jax import lax
from jax.experimental import pallas as pl
from jax.experimental.pallas import tpu as pltpu

N_DEV = 8
E_LOC = 8
T = 2048
D = 512
H = 1024
TC = T // N_DEV


def kernel(x, router_W, route_idx, expert_W):
    def body(x_ref, rw_ref, idx_ref, ew_ref, out_ref,
             wbf_ref, xs_ref, coef_ref, send_buf, recv_buf,
             send_sems, recv_sems):
        d = lax.axis_index("i")
        left = jnp.mod(d - 1, N_DEV)
        right = jnp.mod(d + 1, N_DEV)

        barrier_sem = pltpu.get_barrier_semaphore()
        for nbr in (left, right):
            pl.semaphore_signal(
                barrier_sem, inc=1,
                device_id=(nbr,), device_id_type=pl.DeviceIdType.MESH,
            )
        pl.semaphore_wait(barrier_sem, 2)

        for j in range(E_LOC):
            wbf_ref[pl.ds(D * j, D), :] = ew_ref[j].astype(jnp.bfloat16)

        xb = x_ref[...].astype(jnp.bfloat16)
        scores = jnp.dot(xb, rw_ref[...].astype(jnp.bfloat16),
                         preferred_element_type=jnp.float32)
        idx0 = idx_ref[:, 0:1]
        idx1 = idx_ref[:, 1:2]
        e_iota = lax.broadcasted_iota(jnp.int32, scores.shape, 1)
        s0 = jnp.sum(jnp.where(e_iota == idx0, scores, 0.0), axis=1,
                     keepdims=True)
        s1 = jnp.sum(jnp.where(e_iota == idx1, scores, 0.0), axis=1,
                     keepdims=True)
        g0 = jax.nn.sigmoid(s0 - s1)
        g1 = 1.0 - g0
        gids = d * E_LOC + lax.broadcasted_iota(jnp.int32, (T, E_LOC), 1)
        coef_ref[...] = (jnp.where(idx0 == gids, g0, 0.0)
                         + jnp.where(idx1 == gids, g1, 0.0))

        def partial_chunk(c):
            xc = x_ref[pl.ds(c * TC, TC), :]
            cc = coef_ref[pl.ds(c * TC, TC), :]
            for j in range(E_LOC):
                xs_ref[:, D * j:D * (j + 1)] = (
                    cc[:, j:j + 1] * xc).astype(jnp.bfloat16)
            return jnp.dot(xs_ref[...], wbf_ref[...],
                           preferred_element_type=jnp.float32)

        send_buf[0] = partial_chunk(jnp.mod(d - 1, N_DEV))
        for s in range(N_DEV - 1):
            rdma = pltpu.make_async_remote_copy(
                src_ref=send_buf.at[s],
                dst_ref=recv_buf.at[s],
                send_sem=send_sems.at[s],
                recv_sem=recv_sems.at[s],
                device_id=(right,),
                device_id_type=pl.DeviceIdType.MESH,
            )
            rdma.start()
            rdma.wait()
            c_next = jnp.mod(d - 2 - s, N_DEV)
            if s < N_DEV - 2:
                send_buf[s + 1] = partial_chunk(c_next) + recv_buf[s]
            else:
                out_ref[...] = partial_chunk(c_next) + recv_buf[s]

    return pl.pallas_call(
        body,
        out_shape=jax.ShapeDtypeStruct((TC, H), jnp.float32),
        in_specs=[
            pl.BlockSpec(memory_space=pltpu.VMEM),
            pl.BlockSpec(memory_space=pltpu.VMEM),
            pl.BlockSpec(memory_space=pltpu.VMEM),
            pl.BlockSpec(memory_space=pltpu.VMEM),
        ],
        out_specs=pl.BlockSpec(memory_space=pltpu.VMEM),
        scratch_shapes=[
            pltpu.VMEM((E_LOC * D, H), jnp.bfloat16),
            pltpu.VMEM((TC, E_LOC * D), jnp.bfloat16),
            pltpu.VMEM((T, E_LOC), jnp.float32),
            pltpu.VMEM((N_DEV - 1, TC, H), jnp.float32),
            pltpu.VMEM((N_DEV - 1, TC, H), jnp.float32),
            pltpu.SemaphoreType.DMA((N_DEV - 1,)),
            pltpu.SemaphoreType.DMA((N_DEV - 1,)),
        ],
        compiler_params=pltpu.CompilerParams(collective_id=0),
    )(x, router_W, route_idx, expert_W)


# baseline (device time: 92766 ns/iter reference)
import jax
import jax.numpy as jnp
from jax import lax
from jax.experimental import pallas as pl
from jax.experimental.pallas import tpu as pltpu

N_DEV = 8
E_LOC = 8
T = 2048
D = 512
H = 1024
TC = T // N_DEV


def kernel(x, router_W, route_idx, expert_W):
    def body(x_ref, rw_ref, idx_ref, ew_ref, out_ref,
             wbf_ref, xs_ref, coef_ref, send_buf, recv_buf,
             send_sems, recv_sems):
        d = lax.axis_index("i")
        left = jnp.mod(d - 1, N_DEV)
        right = jnp.mod(d + 1, N_DEV)

        barrier_sem = pltpu.get_barrier_semaphore()
        for nbr in (left, right):
            pl.semaphore_signal(
                barrier_sem, inc=1,
                device_id=(nbr,), device_id_type=pl.DeviceIdType.MESH,
            )
        pl.semaphore_wait(barrier_sem, 2)

        for j in range(E_LOC):
            wbf_ref[pl.ds(D * j, D), :] = ew_ref[j].astype(jnp.bfloat16)

        xb = x_ref[...].astype(jnp.bfloat16)
        scores = jnp.dot(xb, rw_ref[...].astype(jnp.bfloat16),
                         preferred_element_type=jnp.float32)
        idx0 = idx_ref[:, 0:1]
        idx1 = idx_ref[:, 1:2]
        e_iota = lax.broadcasted_iota(jnp.int32, scores.shape, 1)
        s0 = jnp.sum(jnp.where(e_iota == idx0, scores, 0.0), axis=1,
                     keepdims=True)
        s1 = jnp.sum(jnp.where(e_iota == idx1, scores, 0.0), axis=1,
                     keepdims=True)
        g0 = jax.nn.sigmoid(s0 - s1)
        g1 = 1.0 - g0
        gids = d * E_LOC + lax.broadcasted_iota(jnp.int32, (T, E_LOC), 1)
        coef_ref[...] = (jnp.where(idx0 == gids, g0, 0.0)
                         + jnp.where(idx1 == gids, g1, 0.0))

        def partial_chunk(c):
            xc = x_ref[pl.ds(c * TC, TC), :]
            cc = coef_ref[pl.ds(c * TC, TC), :]
            for j in range(E_LOC):
                xs_ref[:, D * j:D * (j + 1)] = (
                    cc[:, j:j + 1] * xc).astype(jnp.bfloat16)
            return jnp.dot(xs_ref[...], wbf_ref[...],
                           preferred_element_type=jnp.float32)

        send_buf[0] = partial_chunk(jnp.mod(d - 1, N_DEV)).astype(jnp.bfloat16)
        for s in range(N_DEV - 1):
            rdma = pltpu.make_async_remote_copy(
                src_ref=send_buf.at[s],
                dst_ref=recv_buf.at[s],
                send_sem=send_sems.at[s],
                recv_sem=recv_sems.at[s],
                device_id=(right,),
                device_id_type=pl.DeviceIdType.MESH,
            )
            rdma.start()
            rdma.wait()
            c_next = jnp.mod(d - 2 - s, N_DEV)
            if s < N_DEV - 2:
                acc = partial_chunk(c_next) + recv_buf[s].astype(jnp.float32)
                send_buf[s + 1] = acc.astype(jnp.bfloat16)
            else:
                out_ref[...] = partial_chunk(c_next) + recv_buf[s].astype(
                    jnp.float32)

    return pl.pallas_call(
        body,
        out_shape=jax.ShapeDtypeStruct((TC, H), jnp.float32),
        in_specs=[
            pl.BlockSpec(memory_space=pltpu.VMEM),
            pl.BlockSpec(memory_space=pltpu.VMEM),
            pl.BlockSpec(memory_space=pltpu.VMEM),
            pl.BlockSpec(memory_space=pltpu.VMEM),
        ],
        out_specs=pl.BlockSpec(memory_space=pltpu.VMEM),
        scratch_shapes=[
            pltpu.VMEM((E_LOC * D, H), jnp.bfloat16),
            pltpu.VMEM((TC, E_LOC * D), jnp.bfloat16),
            pltpu.VMEM((T, E_LOC), jnp.float32),
            pltpu.VMEM((N_DEV - 1, TC, H), jnp.bfloat16),
            pltpu.VMEM((N_DEV - 1, TC, H), jnp.bfloat16),
            pltpu.SemaphoreType.DMA((N_DEV - 1,)),
            pltpu.SemaphoreType.DMA((N_DEV - 1,)),
        ],
        compiler_params=pltpu.CompilerParams(
            collective_id=0,
            vmem_limit_bytes=64 * 1024 * 1024,
        ),
    )(x, router_W, route_idx, expert_W)


# device time: 54283 ns/iter; 1.7089x vs baseline; 1.7089x over previous
import jax
import jax.numpy as jnp
from jax import lax
from jax.experimental import pallas as pl
from jax.experimental.pallas import tpu as pltpu

N_DEV = 8
E_LOC = 8
T = 2048
D = 512
H = 1024
TC = T // N_DEV


def kernel(x, router_W, route_idx, expert_W):
    def body(x_ref, rw_ref, idx_ref, ew_ref, out_ref,
             wbf_ref, xs_ref, coef_ref, send_buf, recv_buf,
             send_sems, recv_sems):
        d = lax.axis_index("i")

        barrier_sem = pltpu.get_barrier_semaphore()
        for k in range(1, N_DEV):
            pl.semaphore_signal(
                barrier_sem, inc=1,
                device_id=(jnp.mod(d + k, N_DEV),),
                device_id_type=pl.DeviceIdType.MESH,
            )
        pl.semaphore_wait(barrier_sem, N_DEV - 1)

        for j in range(E_LOC):
            wbf_ref[pl.ds(D * j, D), :] = ew_ref[j].astype(jnp.bfloat16)

        xb = x_ref[...].astype(jnp.bfloat16)
        scores = jnp.dot(xb, rw_ref[...].astype(jnp.bfloat16),
                         preferred_element_type=jnp.float32)
        idx0 = idx_ref[:, 0:1]
        idx1 = idx_ref[:, 1:2]
        e_iota = lax.broadcasted_iota(jnp.int32, scores.shape, 1)
        s0 = jnp.sum(jnp.where(e_iota == idx0, scores, 0.0), axis=1,
                     keepdims=True)
        s1 = jnp.sum(jnp.where(e_iota == idx1, scores, 0.0), axis=1,
                     keepdims=True)
        g0 = jax.nn.sigmoid(s0 - s1)
        g1 = 1.0 - g0
        gids = d * E_LOC + lax.broadcasted_iota(jnp.int32, (T, E_LOC), 1)
        coef_ref[...] = (jnp.where(idx0 == gids, g0, 0.0)
                         + jnp.where(idx1 == gids, g1, 0.0))

        def partial_chunk(c):
            xc = x_ref[pl.ds(c * TC, TC), :]
            cc = coef_ref[pl.ds(c * TC, TC), :]
            for j in range(E_LOC):
                xs_ref[:, D * j:D * (j + 1)] = (
                    cc[:, j:j + 1] * xc).astype(jnp.bfloat16)
            return jnp.dot(xs_ref[...], wbf_ref[...],
                           preferred_element_type=jnp.float32)

        rdmas = []
        for k in range(1, N_DEV):
            dst = jnp.mod(d + k, N_DEV)
            send_buf[k - 1] = partial_chunk(dst).astype(jnp.bfloat16)
            rdma = pltpu.make_async_remote_copy(
                src_ref=send_buf.at[k - 1],
                dst_ref=recv_buf.at[k - 1],
                send_sem=send_sems.at[k - 1],
                recv_sem=recv_sems.at[k - 1],
                device_id=(dst,),
                device_id_type=pl.DeviceIdType.MESH,
            )
            rdma.start()
            rdmas.append(rdma)

        acc = partial_chunk(d)
        for k in range(1, N_DEV):
            rdmas[k - 1].wait_recv()
            acc = acc + recv_buf[k - 1].astype(jnp.float32)
        out_ref[...] = acc
        for r in rdmas:
            r.wait_send()

    return pl.pallas_call(
        body,
        out_shape=jax.ShapeDtypeStruct((TC, H), jnp.float32),
        in_specs=[
            pl.BlockSpec(memory_space=pltpu.VMEM),
            pl.BlockSpec(memory_space=pltpu.VMEM),
            pl.BlockSpec(memory_space=pltpu.VMEM),
            pl.BlockSpec(memory_space=pltpu.VMEM),
        ],
        out_specs=pl.BlockSpec(memory_space=pltpu.VMEM),
        scratch_shapes=[
            pltpu.VMEM((E_LOC * D, H), jnp.bfloat16),
            pltpu.VMEM((TC, E_LOC * D), jnp.bfloat16),
            pltpu.VMEM((T, E_LOC), jnp.float32),
            pltpu.VMEM((N_DEV - 1, TC, H), jnp.bfloat16),
            pltpu.VMEM((N_DEV - 1, TC, H), jnp.bfloat16),
            pltpu.SemaphoreType.DMA((N_DEV - 1,)),
            pltpu.SemaphoreType.DMA((N_DEV - 1,)),
        ],
        compiler_params=pltpu.CompilerParams(
            collective_id=0,
            vmem_limit_bytes=64 * 1024 * 1024,
        ),
    )(x, router_W, route_idx, expert_W)


# device time: 46485 ns/iter; 1.9956x vs baseline; 1.1678x over previous
import jax
import jax.numpy as jnp
from jax import lax
from jax.experimental import pallas as pl
from jax.experimental.pallas import tpu as pltpu

N_DEV = 8
E_LOC = 8
T = 2048
D = 512
H = 1024
TC = T // N_DEV
CAP = 128


def kernel(x, router_W, route_idx, expert_W):
    def body(x_ref, rw_ref, idx_ref, ew_ref, out_ref,
             wbf_ref, xs_ref, coef_ref, send_buf, recv_buf,
             send_sems, recv_sems):
        d = lax.axis_index("i")

        barrier_sem = pltpu.get_barrier_semaphore()
        for k in range(1, N_DEV):
            pl.semaphore_signal(
                barrier_sem, inc=1,
                device_id=(jnp.mod(d + k, N_DEV),),
                device_id_type=pl.DeviceIdType.MESH,
            )
        pl.semaphore_wait(barrier_sem, N_DEV - 1)

        for j in range(E_LOC):
            wbf_ref[pl.ds(D * j, D), :] = ew_ref[j].astype(jnp.bfloat16)

        xb = x_ref[...].astype(jnp.bfloat16)
        scores = jnp.dot(xb, rw_ref[...].astype(jnp.bfloat16),
                         preferred_element_type=jnp.float32)
        idx0 = idx_ref[:, 0:1]
        idx1 = idx_ref[:, 1:2]
        e_iota = lax.broadcasted_iota(jnp.int32, scores.shape, 1)
        s0 = jnp.sum(jnp.where(e_iota == idx0, scores, 0.0), axis=1,
                     keepdims=True)
        s1 = jnp.sum(jnp.where(e_iota == idx1, scores, 0.0), axis=1,
                     keepdims=True)
        g0 = jax.nn.sigmoid(s0 - s1)
        g1 = 1.0 - g0
        gids = d * E_LOC + lax.broadcasted_iota(jnp.int32, (T, E_LOC), 1)
        coef_ref[...] = (jnp.where(idx0 == gids, g0, 0.0)
                         + jnp.where(idx1 == gids, g1, 0.0))

        def partial_chunk(c):
            xc = x_ref[pl.ds(c * TC, TC), :]
            cc = coef_ref[pl.ds(c * TC, TC), :]
            for j in range(E_LOC):
                xs_ref[:, D * j:D * (j + 1)] = (
                    cc[:, j:j + 1] * xc).astype(jnp.bfloat16)
            return jnp.dot(xs_ref[...], wbf_ref[...],
                           preferred_element_type=jnp.float32)

        slot_iota = lax.broadcasted_iota(jnp.int32, (TC, CAP), 1)
        row_col_bf = lax.broadcasted_iota(
            jnp.int32, (TC, 1), 0).astype(jnp.bfloat16)
        ia = lax.broadcasted_iota(jnp.int32, (TC, TC), 0)
        ib = lax.broadcasted_iota(jnp.int32, (TC, TC), 1)
        tril = (ib < ia).astype(jnp.bfloat16)

        def compress_chunk(c):
            part = partial_chunk(c)
            i0 = idx_ref[pl.ds(c * TC, TC), 0:1]
            i1 = idx_ref[pl.ds(c * TC, TC), 1:2]
            act = jnp.logical_or(i0 // E_LOC == d, i1 // E_LOC == d)
            rank = jnp.dot(tril, act.astype(jnp.bfloat16),
                           preferred_element_type=jnp.float32)
            pt = jnp.logical_and(
                rank.astype(jnp.int32) == slot_iota, act
            ).astype(jnp.bfloat16)
            payload = lax.dot_general(
                pt, part.astype(jnp.bfloat16),
                (((0,), (0,)), ((), ())),
                preferred_element_type=jnp.float32)
            ids = lax.dot_general(
                row_col_bf, pt,
                (((0,), (0,)), ((), ())),
                preferred_element_type=jnp.float32)
            ids_row = jnp.concatenate(
                [ids, jnp.zeros((1, H - CAP), jnp.float32)], axis=1)
            return jnp.concatenate(
                [ids_row, payload], axis=0).astype(jnp.bfloat16)

        rdmas = []
        for k in range(1, N_DEV):
            dst = jnp.mod(d + k, N_DEV)
            send_buf[k - 1] = compress_chunk(dst)
            rdma = pltpu.make_async_remote_copy(
                src_ref=send_buf.at[k - 1],
                dst_ref=recv_buf.at[k - 1],
                send_sem=send_sems.at[k - 1],
                recv_sem=recv_sems.at[k - 1],
                device_id=(dst,),
                device_id_type=pl.DeviceIdType.MESH,
            )
            rdma.start()
            rdmas.append(rdma)

        acc = partial_chunk(d)
        recv_row_iota = lax.broadcasted_iota(jnp.int32, (TC, CAP), 0)
        for k in range(1, N_DEV):
            rdmas[k - 1].wait_recv()
            ids = recv_buf[k - 1, 0:1, 0:CAP].astype(jnp.int32)
            scatter = (recv_row_iota == ids).astype(jnp.bfloat16)
            acc = acc + jnp.dot(scatter, recv_buf[k - 1, 1:1 + CAP, :],
                                preferred_element_type=jnp.float32)
        out_ref[...] = acc
        for r in rdmas:
            r.wait_send()

    return pl.pallas_call(
        body,
        out_shape=jax.ShapeDtypeStruct((TC, H), jnp.float32),
        in_specs=[
            pl.BlockSpec(memory_space=pltpu.VMEM),
            pl.BlockSpec(memory_space=pltpu.VMEM),
            pl.BlockSpec(memory_space=pltpu.VMEM),
            pl.BlockSpec(memory_space=pltpu.VMEM),
        ],
        out_specs=pl.BlockSpec(memory_space=pltpu.VMEM),
        scratch_shapes=[
            pltpu.VMEM((E_LOC * D, H), jnp.bfloat16),
            pltpu.VMEM((TC, E_LOC * D), jnp.bfloat16),
            pltpu.VMEM((T, E_LOC), jnp.float32),
            pltpu.VMEM((N_DEV - 1, 1 + CAP, H), jnp.bfloat16),
            pltpu.VMEM((N_DEV - 1, 1 + CAP, H), jnp.bfloat16),
            pltpu.SemaphoreType.DMA((N_DEV - 1,)),
            pltpu.SemaphoreType.DMA((N_DEV - 1,)),
        ],
        compiler_params=pltpu.CompilerParams(
            collective_id=0,
            vmem_limit_bytes=64 * 1024 * 1024,
        ),
    )(x, router_W, route_idx, expert_W)


# device time: 42064 ns/iter; 2.2054x vs baseline; 1.1051x over previous
import jax
import jax.numpy as jnp
from jax import lax
from jax.experimental import pallas as pl
from jax.experimental.pallas import tpu as pltpu

N_DEV = 8
E_LOC = 8
T = 2048
D = 512
H = 1024
TC = T // N_DEV
CAP = 128


def kernel(x, router_W, route_idx, expert_W):
    def body(x_ref, rw_ref, idx_ref, ew_ref, out_ref,
             wbf_ref, xs_ref, xsc_ref, coef_ref, send_buf, recv_buf,
             send_sems, recv_sems):
        d = lax.axis_index("i")

        barrier_sem = pltpu.get_barrier_semaphore()
        for k in range(1, N_DEV):
            pl.semaphore_signal(
                barrier_sem, inc=1,
                device_id=(jnp.mod(d + k, N_DEV),),
                device_id_type=pl.DeviceIdType.MESH,
            )
        pl.semaphore_wait(barrier_sem, N_DEV - 1)

        for j in range(E_LOC):
            wbf_ref[pl.ds(D * j, D), :] = ew_ref[j].astype(jnp.bfloat16)

        xb = x_ref[...].astype(jnp.bfloat16)
        scores = jnp.dot(xb, rw_ref[...].astype(jnp.bfloat16),
                         preferred_element_type=jnp.float32)
        idx0 = idx_ref[:, 0:1]
        idx1 = idx_ref[:, 1:2]
        e_iota = lax.broadcasted_iota(jnp.int32, scores.shape, 1)
        s0 = jnp.sum(jnp.where(e_iota == idx0, scores, 0.0), axis=1,
                     keepdims=True)
        s1 = jnp.sum(jnp.where(e_iota == idx1, scores, 0.0), axis=1,
                     keepdims=True)
        g0 = jax.nn.sigmoid(s0 - s1)
        g1 = 1.0 - g0
        gids = d * E_LOC + lax.broadcasted_iota(jnp.int32, (T, E_LOC), 1)
        coef_ref[...] = (jnp.where(idx0 == gids, g0, 0.0)
                         + jnp.where(idx1 == gids, g1, 0.0))

        def partial_chunk(c):
            xc = x_ref[pl.ds(c * TC, TC), :]
            cc = coef_ref[pl.ds(c * TC, TC), :]
            for j in range(E_LOC):
                xs_ref[:, D * j:D * (j + 1)] = (
                    cc[:, j:j + 1] * xc).astype(jnp.bfloat16)
            return jnp.dot(xs_ref[...], wbf_ref[...],
                           preferred_element_type=jnp.float32)

        slot_iota = lax.broadcasted_iota(jnp.int32, (TC, CAP), 1)
        row_col_bf = lax.broadcasted_iota(
            jnp.int32, (TC, 1), 0).astype(jnp.bfloat16)
        ia = lax.broadcasted_iota(jnp.int32, (TC, TC), 0)
        ib = lax.broadcasted_iota(jnp.int32, (TC, TC), 1)
        tril = (ib < ia).astype(jnp.bfloat16)

        def compress_chunk(c):
            i0 = idx_ref[pl.ds(c * TC, TC), 0:1]
            i1 = idx_ref[pl.ds(c * TC, TC), 1:2]
            act = jnp.logical_or(i0 // E_LOC == d, i1 // E_LOC == d)
            rank = jnp.dot(tril, act.astype(jnp.bfloat16),
                           preferred_element_type=jnp.float32)
            pt = jnp.logical_and(
                rank.astype(jnp.int32) == slot_iota, act
            ).astype(jnp.bfloat16)
            cc = coef_ref[pl.ds(c * TC, TC), :].astype(jnp.bfloat16)
            xc = x_ref[pl.ds(c * TC, TC), :].astype(jnp.bfloat16)
            for j in range(E_LOC):
                g = pt * cc[:, j:j + 1]
                xsc_ref[:, D * j:D * (j + 1)] = lax.dot_general(
                    g, xc, (((0,), (0,)), ((), ())),
                    preferred_element_type=jnp.float32).astype(jnp.bfloat16)
            payload = jnp.dot(xsc_ref[...], wbf_ref[...],
                              preferred_element_type=jnp.float32)
            ids = lax.dot_general(
                row_col_bf, pt,
                (((0,), (0,)), ((), ())),
                preferred_element_type=jnp.float32)
            ids_row = jnp.concatenate(
                [ids, jnp.zeros((1, H - CAP), jnp.float32)], axis=1)
            return jnp.concatenate(
                [ids_row, payload], axis=0).astype(jnp.bfloat16)

        rdmas = []
        for k in range(1, N_DEV):
            dst = jnp.mod(d + k, N_DEV)
            send_buf[k - 1] = compress_chunk(dst)
            rdma = pltpu.make_async_remote_copy(
                src_ref=send_buf.at[k - 1],
                dst_ref=recv_buf.at[k - 1],
                send_sem=send_sems.at[k - 1],
                recv_sem=recv_sems.at[k - 1],
                device_id=(dst,),
                device_id_type=pl.DeviceIdType.MESH,
            )
            rdma.start()
            rdmas.append(rdma)

        acc = partial_chunk(d)
        recv_row_iota = lax.broadcasted_iota(jnp.int32, (TC, CAP), 0)
        for k in range(1, N_DEV):
            rdmas[k - 1].wait_recv()
            ids = recv_buf[k - 1, 0:1, 0:CAP].astype(jnp.int32)
            scatter = (recv_row_iota == ids).astype(jnp.bfloat16)
            acc = acc + jnp.dot(scatter, recv_buf[k - 1, 1:1 + CAP, :],
                                preferred_element_type=jnp.float32)
        out_ref[...] = acc
        for r in rdmas:
            r.wait_send()

    return pl.pallas_call(
        body,
        out_shape=jax.ShapeDtypeStruct((TC, H), jnp.float32),
        in_specs=[
            pl.BlockSpec(memory_space=pltpu.VMEM),
            pl.BlockSpec(memory_space=pltpu.VMEM),
            pl.BlockSpec(memory_space=pltpu.VMEM),
            pl.BlockSpec(memory_space=pltpu.VMEM),
        ],
        out_specs=pl.BlockSpec(memory_space=pltpu.VMEM),
        scratch_shapes=[
            pltpu.VMEM((E_LOC * D, H), jnp.bfloat16),
            pltpu.VMEM((TC, E_LOC * D), jnp.bfloat16),
            pltpu.VMEM((CAP, E_LOC * D), jnp.bfloat16),
            pltpu.VMEM((T, E_LOC), jnp.float32),
            pltpu.VMEM((N_DEV - 1, 1 + CAP, H), jnp.bfloat16),
            pltpu.VMEM((N_DEV - 1, 1 + CAP, H), jnp.bfloat16),
            pltpu.SemaphoreType.DMA((N_DEV - 1,)),
            pltpu.SemaphoreType.DMA((N_DEV - 1,)),
        ],
        compiler_params=pltpu.CompilerParams(
            collective_id=0,
            vmem_limit_bytes=64 * 1024 * 1024,
        ),
    )(x, router_W, route_idx, expert_W)


# device time: 41862 ns/iter; 2.2160x vs baseline; 1.0048x over previous
import jax
import jax.numpy as jnp
from jax import lax
from jax.experimental import pallas as pl
from jax.experimental.pallas import tpu as pltpu

N_DEV = 8
E_LOC = 8
T = 2048
D = 512
H = 1024
TC = T // N_DEV
CAP = 128


def kernel(x, router_W, route_idx, expert_W):
    def body(x_ref, rw_ref, idx_ref, ew_ref, out_ref,
             wbf_ref, xsc_ref, coef_ref, send_buf, recv_buf,
             send_sems, recv_sems):
        d = lax.axis_index("i")

        barrier_sem = pltpu.get_barrier_semaphore()
        for k in range(1, N_DEV):
            pl.semaphore_signal(
                barrier_sem, inc=1,
                device_id=(jnp.mod(d + k, N_DEV),),
                device_id_type=pl.DeviceIdType.MESH,
            )
        pl.semaphore_wait(barrier_sem, N_DEV - 1)

        for j in range(E_LOC):
            wbf_ref[pl.ds(D * j, D), :] = ew_ref[j].astype(jnp.bfloat16)

        xb = x_ref[...].astype(jnp.bfloat16)
        scores = jnp.dot(xb, rw_ref[...].astype(jnp.bfloat16),
                         preferred_element_type=jnp.float32)
        idx0 = idx_ref[:, 0:1]
        idx1 = idx_ref[:, 1:2]
        e_iota = lax.broadcasted_iota(jnp.int32, scores.shape, 1)
        s0 = jnp.sum(jnp.where(e_iota == idx0, scores, 0.0), axis=1,
                     keepdims=True)
        s1 = jnp.sum(jnp.where(e_iota == idx1, scores, 0.0), axis=1,
                     keepdims=True)
        g0 = jax.nn.sigmoid(s0 - s1)
        g1 = 1.0 - g0
        gids = d * E_LOC + lax.broadcasted_iota(jnp.int32, (T, E_LOC), 1)
        coef_ref[...] = (jnp.where(idx0 == gids, g0, 0.0)
                         + jnp.where(idx1 == gids, g1, 0.0))

        slot_iota = lax.broadcasted_iota(jnp.int32, (TC, CAP), 1)
        row_col_bf = lax.broadcasted_iota(
            jnp.int32, (TC, 1), 0).astype(jnp.bfloat16)
        ia = lax.broadcasted_iota(jnp.int32, (TC, TC), 0)
        ib = lax.broadcasted_iota(jnp.int32, (TC, TC), 1)
        tril = (ib < ia).astype(jnp.bfloat16)

        def build_pt(c):
            i0 = idx_ref[pl.ds(c * TC, TC), 0:1]
            i1 = idx_ref[pl.ds(c * TC, TC), 1:2]
            act = jnp.logical_or(i0 // E_LOC == d, i1 // E_LOC == d)
            rank = jnp.dot(tril, act.astype(jnp.bfloat16),
                           preferred_element_type=jnp.float32)
            return jnp.logical_and(
                rank.astype(jnp.int32) == slot_iota, act
            ).astype(jnp.bfloat16)

        def gather_into(c, pt, half):
            cc = coef_ref[pl.ds(c * TC, TC), :].astype(jnp.bfloat16)
            xc = x_ref[pl.ds(c * TC, TC), :].astype(jnp.bfloat16)
            for j in range(E_LOC):
                g = pt * cc[:, j:j + 1]
                xsc_ref[CAP * half:CAP * (half + 1), D * j:D * (j + 1)] = (
                    lax.dot_general(
                        g, xc, (((0,), (0,)), ((), ())),
                        preferred_element_type=jnp.float32
                    ).astype(jnp.bfloat16))

        def make_msg(pt, payload):
            ids = lax.dot_general(
                row_col_bf, pt,
                (((0,), (0,)), ((), ())),
                preferred_element_type=jnp.float32)
            ids_row = jnp.concatenate(
                [ids, jnp.zeros((1, H - CAP), jnp.float32)], axis=1)
            return jnp.concatenate(
                [ids_row, payload], axis=0).astype(jnp.bfloat16)

        rdmas = [None] * (N_DEV - 1)
        acc = None
        ks = [1, 2, 3, 4, 5, 6, 7, 0]
        for p in range(N_DEV // 2):
            pair = ks[2 * p:2 * p + 2]
            pts = []
            for half, k in enumerate(pair):
                c = jnp.mod(d + k, N_DEV)
                pt = build_pt(c)
                gather_into(c, pt, half)
                pts.append(pt)
            y = jnp.dot(xsc_ref[...], wbf_ref[...],
                        preferred_element_type=jnp.float32)
            for half, k in enumerate(pair):
                payload = y[CAP * half:CAP * (half + 1), :]
                if k == 0:
                    acc = jnp.dot(pts[half], payload.astype(jnp.bfloat16),
                                  preferred_element_type=jnp.float32)
                    continue
                send_buf[k - 1] = make_msg(pts[half], payload)
                rdma = pltpu.make_async_remote_copy(
                    src_ref=send_buf.at[k - 1],
                    dst_ref=recv_buf.at[k - 1],
                    send_sem=send_sems.at[k - 1],
                    recv_sem=recv_sems.at[k - 1],
                    device_id=(jnp.mod(d + k, N_DEV),),
                    device_id_type=pl.DeviceIdType.MESH,
                )
                rdma.start()
                rdmas[k - 1] = rdma

        recv_row_iota = lax.broadcasted_iota(jnp.int32, (TC, CAP), 0)
        for k in range(1, N_DEV):
            rdmas[k - 1].wait_recv()
            ids = recv_buf[k - 1, 0:1, 0:CAP].astype(jnp.int32)
            scatter = (recv_row_iota == ids).astype(jnp.bfloat16)
            acc = acc + jnp.dot(scatter, recv_buf[k - 1, 1:1 + CAP, :],
                                preferred_element_type=jnp.float32)
        out_ref[...] = acc
        for r in rdmas:
            r.wait_send()

    return pl.pallas_call(
        body,
        out_shape=jax.ShapeDtypeStruct((TC, H), jnp.float32),
        in_specs=[
            pl.BlockSpec(memory_space=pltpu.VMEM),
            pl.BlockSpec(memory_space=pltpu.VMEM),
            pl.BlockSpec(memory_space=pltpu.VMEM),
            pl.BlockSpec(memory_space=pltpu.VMEM),
        ],
        out_specs=pl.BlockSpec(memory_space=pltpu.VMEM),
        scratch_shapes=[
            pltpu.VMEM((E_LOC * D, H), jnp.bfloat16),
            pltpu.VMEM((2 * CAP, E_LOC * D), jnp.bfloat16),
            pltpu.VMEM((T, E_LOC), jnp.float32),
            pltpu.VMEM((N_DEV - 1, 1 + CAP, H), jnp.bfloat16),
            pltpu.VMEM((N_DEV - 1, 1 + CAP, H), jnp.bfloat16),
            pltpu.SemaphoreType.DMA((N_DEV - 1,)),
            pltpu.SemaphoreType.DMA((N_DEV - 1,)),
        ],
        compiler_params=pltpu.CompilerParams(
            collective_id=0,
            vmem_limit_bytes=64 * 1024 * 1024,
        ),
    )(x, router_W, route_idx, expert_W)


# device time: 41040 ns/iter; 2.2604x vs baseline; 1.0200x over previous
import jax
import jax.numpy as jnp
from jax import lax
from jax.experimental import pallas as pl
from jax.experimental.pallas import tpu as pltpu

N_DEV = 8
E_LOC = 8
T = 2048
D = 512
H = 1024
TC = T // N_DEV
CAP = 128


def kernel(x, router_W, route_idx, expert_W):
    def body(x_ref, rw_ref, idx_ref, ew_ref, out_ref,
             wbf_ref, xsc_ref, coef_ref, send_buf, recv_buf,
             send_sems, recv_sems):
        d = lax.axis_index("i")

        with jax.named_scope("entrybarrier"):
            barrier_sem = pltpu.get_barrier_semaphore()
            for k in range(1, N_DEV):
                pl.semaphore_signal(
                    barrier_sem, inc=1,
                    device_id=(jnp.mod(d + k, N_DEV),),
                    device_id_type=pl.DeviceIdType.MESH,
                )
            pl.semaphore_wait(barrier_sem, N_DEV - 1)

        with jax.named_scope("precast"):
            for j in range(E_LOC):
                wbf_ref[pl.ds(D * j, D), :] = ew_ref[j].astype(jnp.bfloat16)

        with jax.named_scope("gating"):
            xb = x_ref[...].astype(jnp.bfloat16)
            scores = jnp.dot(xb, rw_ref[...].astype(jnp.bfloat16),
                             preferred_element_type=jnp.float32)
            idx0 = idx_ref[:, 0:1]
            idx1 = idx_ref[:, 1:2]
            e_iota = lax.broadcasted_iota(jnp.int32, scores.shape, 1)
            s0 = jnp.sum(jnp.where(e_iota == idx0, scores, 0.0), axis=1,
                         keepdims=True)
            s1 = jnp.sum(jnp.where(e_iota == idx1, scores, 0.0), axis=1,
                         keepdims=True)
            g0 = jax.nn.sigmoid(s0 - s1)
            g1 = 1.0 - g0
            gids = d * E_LOC + lax.broadcasted_iota(jnp.int32, (T, E_LOC), 1)
            coef_ref[...] = (jnp.where(idx0 == gids, g0, 0.0)
                             + jnp.where(idx1 == gids, g1, 0.0))

        slot_iota = lax.broadcasted_iota(jnp.int32, (TC, CAP), 1)
        row_col_bf = lax.broadcasted_iota(
            jnp.int32, (TC, 1), 0).astype(jnp.bfloat16)
        ia = lax.broadcasted_iota(jnp.int32, (TC, TC), 0)
        ib = lax.broadcasted_iota(jnp.int32, (TC, TC), 1)
        tril = (ib < ia).astype(jnp.bfloat16)

        def build_pt(c):
            i0 = idx_ref[pl.ds(c * TC, TC), 0:1]
            i1 = idx_ref[pl.ds(c * TC, TC), 1:2]
            act = jnp.logical_or(i0 // E_LOC == d, i1 // E_LOC == d)
            rank = jnp.dot(tril, act.astype(jnp.bfloat16),
                           preferred_element_type=jnp.float32)
            return jnp.logical_and(
                rank.astype(jnp.int32) == slot_iota, act
            ).astype(jnp.bfloat16)

        def gather_into(c, pt, half):
            cc = coef_ref[pl.ds(c * TC, TC), :]
            xc = x_ref[pl.ds(c * TC, TC), :].astype(jnp.bfloat16)
            xg = lax.dot_general(
                pt, xc, (((0,), (0,)), ((), ())),
                preferred_element_type=jnp.float32)
            ccomp = lax.dot_general(
                pt, cc.astype(jnp.bfloat16), (((0,), (0,)), ((), ())),
                preferred_element_type=jnp.float32)
            for j in range(E_LOC):
                xsc_ref[CAP * half:CAP * (half + 1), D * j:D * (j + 1)] = (
                    ccomp[:, j:j + 1] * xg).astype(jnp.bfloat16)

        def make_msg(pt, payload):
            ids = lax.dot_general(
                row_col_bf, pt,
                (((0,), (0,)), ((), ())),
                preferred_element_type=jnp.float32)
            ids_row = jnp.concatenate(
                [ids, jnp.zeros((1, H - CAP), jnp.float32)], axis=1)
            return jnp.concatenate(
                [ids_row, payload], axis=0).astype(jnp.bfloat16)

        rdmas = [None] * (N_DEV - 1)
        acc = None
        ks = [1, 2, 3, 4, 5, 6, 7, 0]
        for p in range(N_DEV // 2):
            with jax.named_scope(f"pair{p}"):
                pair = ks[2 * p:2 * p + 2]
                pts = []
                for half, k in enumerate(pair):
                    c = jnp.mod(d + k, N_DEV)
                    pt = build_pt(c)
                    gather_into(c, pt, half)
                    pts.append(pt)
                y = jnp.dot(xsc_ref[...], wbf_ref[...],
                            preferred_element_type=jnp.float32)
                for half, k in enumerate(pair):
                    payload = y[CAP * half:CAP * (half + 1), :]
                    if k == 0:
                        acc = jnp.dot(pts[half], payload.astype(jnp.bfloat16),
                                      preferred_element_type=jnp.float32)
                        continue
                    send_buf[k - 1] = make_msg(pts[half], payload)
                    rdma = pltpu.make_async_remote_copy(
                        src_ref=send_buf.at[k - 1],
                        dst_ref=recv_buf.at[k - 1],
                        send_sem=send_sems.at[k - 1],
                        recv_sem=recv_sems.at[k - 1],
                        device_id=(jnp.mod(d + k, N_DEV),),
                        device_id_type=pl.DeviceIdType.MESH,
                    )
                    rdma.start()
                    rdmas[k - 1] = rdma

        with jax.named_scope("waitadd"):
            recv_row_iota = lax.broadcasted_iota(jnp.int32, (TC, CAP), 0)
            for k in range(1, N_DEV):
                rdmas[k - 1].wait_recv()
                ids = recv_buf[k - 1, 0:1, 0:CAP].astype(jnp.int32)
                scatter = (recv_row_iota == ids).astype(jnp.bfloat16)
                acc = acc + jnp.dot(scatter, recv_buf[k - 1, 1:1 + CAP, :],
                                    preferred_element_type=jnp.float32)
            out_ref[...] = acc
            for r in rdmas:
                r.wait_send()

    return pl.pallas_call(
        body,
        out_shape=jax.ShapeDtypeStruct((TC, H), jnp.float32),
        in_specs=[
            pl.BlockSpec(memory_space=pltpu.VMEM),
            pl.BlockSpec(memory_space=pltpu.VMEM),
            pl.BlockSpec(memory_space=pltpu.VMEM),
            pl.BlockSpec(memory_space=pltpu.VMEM),
        ],
        out_specs=pl.BlockSpec(memory_space=pltpu.VMEM),
        scratch_shapes=[
            pltpu.VMEM((E_LOC * D, H), jnp.bfloat16),
            pltpu.VMEM((2 * CAP, E_LOC * D), jnp.bfloat16),
            pltpu.VMEM((T, E_LOC), jnp.float32),
            pltpu.VMEM((N_DEV - 1, 1 + CAP, H), jnp.bfloat16),
            pltpu.VMEM((N_DEV - 1, 1 + CAP, H), jnp.bfloat16),
            pltpu.SemaphoreType.DMA((N_DEV - 1,)),
            pltpu.SemaphoreType.DMA((N_DEV - 1,)),
        ],
        compiler_params=pltpu.CompilerParams(
            collective_id=0,
            vmem_limit_bytes=64 * 1024 * 1024,
        ),
    )(x, router_W, route_idx, expert_W)


# device time: 37901 ns/iter; 2.4476x vs baseline; 1.0828x over previous
import jax
import jax.numpy as jnp
from jax import lax
from jax.experimental import pallas as pl
from jax.experimental.pallas import tpu as pltpu

N_DEV = 8
E_LOC = 8
T = 2048
D = 512
H = 1024
TC = T // N_DEV
CAP = 128
ABLATE_NO_COMM = False


def kernel(x, router_W, route_idx, expert_W):
    def body(x_ref, rw_ref, idx_ref, ew_ref, out_ref,
             wbf_ref, xsc_ref, coef_ref, send_buf, recv_buf, ew_vmem,
             send_sems, recv_sems, load_sems):
        d = lax.axis_index("i")

        ldmas = []
        for j in range(E_LOC):
            ldma = pltpu.make_async_copy(
                ew_ref.at[j], ew_vmem.at[j], load_sems.at[j])
            ldma.start()
            ldmas.append(ldma)

        with jax.named_scope("entrybarrier"):
            if not ABLATE_NO_COMM:
                barrier_sem = pltpu.get_barrier_semaphore()
                for k in range(1, N_DEV):
                    pl.semaphore_signal(
                        barrier_sem, inc=1,
                        device_id=(jnp.mod(d + k, N_DEV),),
                        device_id_type=pl.DeviceIdType.MESH,
                    )
                pl.semaphore_wait(barrier_sem, N_DEV - 1)

        with jax.named_scope("gating"):
            xb = x_ref[...].astype(jnp.bfloat16)
            scores = jnp.dot(xb, rw_ref[...].astype(jnp.bfloat16),
                             preferred_element_type=jnp.float32)
            idx0 = idx_ref[:, 0:1]
            idx1 = idx_ref[:, 1:2]
            e_iota = lax.broadcasted_iota(jnp.int32, scores.shape, 1)
            s0 = jnp.sum(jnp.where(e_iota == idx0, scores, 0.0), axis=1,
                         keepdims=True)
            s1 = jnp.sum(jnp.where(e_iota == idx1, scores, 0.0), axis=1,
                         keepdims=True)
            g0 = jax.nn.sigmoid(s0 - s1)
            g1 = 1.0 - g0
            gids = d * E_LOC + lax.broadcasted_iota(jnp.int32, (T, E_LOC), 1)
            coef_ref[...] = (jnp.where(idx0 == gids, g0, 0.0)
                             + jnp.where(idx1 == gids, g1, 0.0))

        slot_iota = lax.broadcasted_iota(jnp.int32, (TC, CAP), 1)
        row_col_bf = lax.broadcasted_iota(
            jnp.int32, (TC, 1), 0).astype(jnp.bfloat16)
        ia = lax.broadcasted_iota(jnp.int32, (TC, TC), 0)
        ib = lax.broadcasted_iota(jnp.int32, (TC, TC), 1)
        tril = (ib < ia).astype(jnp.bfloat16)

        def build_pt(c):
            i0 = idx_ref[pl.ds(c * TC, TC), 0:1]
            i1 = idx_ref[pl.ds(c * TC, TC), 1:2]
            act = jnp.logical_or(i0 // E_LOC == d, i1 // E_LOC == d)
            rank = jnp.dot(tril, act.astype(jnp.bfloat16),
                           preferred_element_type=jnp.float32)
            return jnp.logical_and(
                rank.astype(jnp.int32) == slot_iota, act
            ).astype(jnp.bfloat16)

        def gather_into(c, pt, half):
            cc = coef_ref[pl.ds(c * TC, TC), :]
            xc = x_ref[pl.ds(c * TC, TC), :].astype(jnp.bfloat16)
            xg = lax.dot_general(
                pt, xc, (((0,), (0,)), ((), ())),
                preferred_element_type=jnp.float32)
            ccomp = lax.dot_general(
                pt, cc.astype(jnp.bfloat16), (((0,), (0,)), ((), ())),
                preferred_element_type=jnp.float32)
            for j in range(E_LOC):
                xsc_ref[CAP * half:CAP * (half + 1), D * j:D * (j + 1)] = (
                    ccomp[:, j:j + 1] * xg).astype(jnp.bfloat16)

        def make_msg(pt, payload):
            ids = lax.dot_general(
                row_col_bf, pt,
                (((0,), (0,)), ((), ())),
                preferred_element_type=jnp.float32)
            ids_row = jnp.concatenate(
                [ids, jnp.zeros((1, H - CAP), jnp.float32)], axis=1)
            return jnp.concatenate(
                [ids_row, payload], axis=0).astype(jnp.bfloat16)

        rdmas = [None] * (N_DEV - 1)
        acc = None
        ks = [1, 2, 3, 4, 5, 6, 7, 0]
        for p in range(N_DEV // 2):
            with jax.named_scope(f"pair{p}"):
                pair = ks[2 * p:2 * p + 2]
                pts = []
                for half, k in enumerate(pair):
                    c = jnp.mod(d + k, N_DEV)
                    pt = build_pt(c)
                    gather_into(c, pt, half)
                    pts.append(pt)
                if p == 0:
                    with jax.named_scope("precast"):
                        for j in range(E_LOC):
                            ldmas[j].wait()
                            wbf_ref[pl.ds(D * j, D), :] = (
                                ew_vmem[j].astype(jnp.bfloat16))
                y = jnp.dot(xsc_ref[...], wbf_ref[...],
                            preferred_element_type=jnp.float32)
                for half, k in enumerate(pair):
                    payload = y[CAP * half:CAP * (half + 1), :]
                    if k == 0:
                        acc = jnp.dot(pts[half], payload.astype(jnp.bfloat16),
                                      preferred_element_type=jnp.float32)
                        continue
                    send_buf[k - 1] = make_msg(pts[half], payload)
                    if not ABLATE_NO_COMM:
                        rdma = pltpu.make_async_remote_copy(
                            src_ref=send_buf.at[k - 1],
                            dst_ref=recv_buf.at[k - 1],
                            send_sem=send_sems.at[k - 1],
                            recv_sem=recv_sems.at[k - 1],
                            device_id=(jnp.mod(d + k, N_DEV),),
                            device_id_type=pl.DeviceIdType.MESH,
                        )
                        rdma.start()
                        rdmas[k - 1] = rdma

        with jax.named_scope("waitadd"):
            recv_row_iota = lax.broadcasted_iota(jnp.int32, (TC, CAP), 0)
            for k in range(1, N_DEV):
                if ABLATE_NO_COMM:
                    continue
                rdmas[k - 1].wait_recv()
                ids = recv_buf[k - 1, 0:1, 0:CAP].astype(jnp.int32)
                scatter = (recv_row_iota == ids).astype(jnp.bfloat16)
                acc = acc + jnp.dot(scatter, recv_buf[k - 1, 1:1 + CAP, :],
                                    preferred_element_type=jnp.float32)
            out_ref[...] = acc
            for r in rdmas:
                if r is not None:
                    r.wait_send()

    return pl.pallas_call(
        body,
        out_shape=jax.ShapeDtypeStruct((TC, H), jnp.float32),
        in_specs=[
            pl.BlockSpec(memory_space=pltpu.VMEM),
            pl.BlockSpec(memory_space=pltpu.VMEM),
            pl.BlockSpec(memory_space=pltpu.VMEM),
            pl.BlockSpec(memory_space=pl.ANY),
        ],
        out_specs=pl.BlockSpec(memory_space=pltpu.VMEM),
        scratch_shapes=[
            pltpu.VMEM((E_LOC * D, H), jnp.bfloat16),
            pltpu.VMEM((2 * CAP, E_LOC * D), jnp.bfloat16),
            pltpu.VMEM((T, E_LOC), jnp.float32),
            pltpu.VMEM((N_DEV - 1, 1 + CAP, H), jnp.bfloat16),
            pltpu.VMEM((N_DEV - 1, 1 + CAP, H), jnp.bfloat16),
            pltpu.VMEM((E_LOC, D, H), jnp.float32),
            pltpu.SemaphoreType.DMA((N_DEV - 1,)),
            pltpu.SemaphoreType.DMA((N_DEV - 1,)),
            pltpu.SemaphoreType.DMA((E_LOC,)),
        ],
        compiler_params=pltpu.CompilerParams(
            collective_id=None if ABLATE_NO_COMM else 0,
            vmem_limit_bytes=64 * 1024 * 1024,
        ),
    )(x, router_W, route_idx, expert_W)
